# baseline (speedup 1.0000x reference)
import jax
import jax.numpy as jnp
from jax.experimental import pallas as pl
from jax.experimental.pallas import tpu as pltpu

NZ = 34
E = 8
MAX_QT = 15
MAX_SLI = 18
SCALE = 1.0
EP = 72
FX = 80
BT = 8192


def _fold_weights(wb, bb, cq, cs, iq, isl):
    npred = MAX_QT + MAX_SLI
    coef_cat = jnp.concatenate([cq, cs], axis=1)
    int_cat = jnp.concatenate([iq, isl], axis=1)
    A = coef_cat[:, :, :npred]
    W_sel = jnp.concatenate([wb[:MAX_QT], wb[NZ:NZ + MAX_SLI]], axis=0)
    b_sel = jnp.concatenate([bb[:, :MAX_QT], bb[:, NZ:NZ + MAX_SLI]], axis=1)
    D = jnp.concatenate([
        coef_cat[:, :, npred + 1:npred + 1 + NZ],
        coef_cat[:, :, npred + 1 + NZ:npred + 1 + 2 * NZ],
        coef_cat[:, :, npred:npred + 1],
        coef_cat[:, :, npred + 1 + 2 * NZ:],
    ], axis=2)
    M = jax.lax.dot_general(
        A.reshape(E * 2 * NZ, npred), W_sel,
        (((1,), (0,)), ((), ())),
        preferred_element_type=jnp.float32).reshape(E, 2 * NZ, 70)
    R = wb[None] + SCALE * (M + D)
    g = bb + SCALE * (jnp.sum(A * b_sel[:, None, :], axis=-1) + int_cat)
    return R, g


def _routed_kernel(qt_ref, sli_ref, sst_ref, sol_ref, eta_ref,
                   wb_ref, bb_ref, cq_ref, iq_ref, cs_ref, is_ref,
                   out_ref, gs_ref):
    @pl.when(pl.program_id(0) == 0)
    def _fold():
        R, g = _fold_weights(wb_ref[...], bb_ref[...], cq_ref[...],
                             cs_ref[...], iq_ref[...], is_ref[...])
        gs_ref[...] = jnp.zeros((E * EP, FX), jnp.float32)
        for e in range(E):
            gs_ref[e * EP:e * EP + 2 * NZ, 0:NZ] = R[e, :, 0:NZ]
            gs_ref[e * EP:e * EP + 2 * NZ, 40:40 + NZ] = R[e, :, NZ:2 * NZ]
            gs_ref[e * EP:e * EP + 2 * NZ, 74:76] = R[e, :, 2 * NZ:]
            gs_ref[e * EP:e * EP + 2 * NZ, 76:77] = g[e, :, None]

    bt = qt_ref.shape[1] * qt_ref.shape[2]
    x = jnp.concatenate([
        qt_ref[...].reshape(NZ, bt),
        jnp.zeros((40 - NZ, bt), jnp.float32),
        sli_ref[...].reshape(NZ, bt),
        sst_ref[...].reshape(1, bt),
        sol_ref[...].reshape(1, bt),
        jnp.ones((1, bt), jnp.float32),
        jnp.zeros((FX - 77, bt), jnp.float32),
    ], axis=0)
    y = jax.lax.dot_general(
        gs_ref[...].astype(jnp.bfloat16), x.astype(jnp.bfloat16),
        (((1,), (0,)), ((), ())),
        preferred_element_type=jnp.float32)
    eta = eta_ref[...].reshape(1, bt)
    b0 = (eta & 1) == 1
    b1 = (eta & 2) == 2
    b2 = (eta & 4) == 4
    z = [jnp.where(b0, y[(2 * k + 1) * EP:(2 * k + 2) * EP, :],
                   y[2 * k * EP:(2 * k + 1) * EP, :]) for k in range(4)]
    q0 = jnp.where(b1, z[1], z[0])
    q1 = jnp.where(b1, z[3], z[2])
    acc = jnp.where(b2, q1, q0)
    out_ref[...] = acc[:2 * NZ, :].reshape(out_ref.shape)


def kernel(QT, SLI, SST, SOLIN, layer_mass, eta, W_base, b_base,
           coef_qt, int_qt, coef_sli, int_sli):
    nz, h, w = QT.shape
    n = h * w
    hb = BT // w
    out = pl.pallas_call(
        _routed_kernel,
        grid=(n // BT,),
        in_specs=[
            pl.BlockSpec((nz, hb, w), lambda i: (0, i, 0)),
            pl.BlockSpec((nz, hb, w), lambda i: (0, i, 0)),
            pl.BlockSpec((hb, w), lambda i: (i, 0)),
            pl.BlockSpec((hb, w), lambda i: (i, 0)),
            pl.BlockSpec((hb, w), lambda i: (i, 0)),
            pl.BlockSpec((2 * nz, 2 * nz + 2), lambda i: (0, 0)),
            pl.BlockSpec((1, 2 * nz), lambda i: (0, 0)),
            pl.BlockSpec((E, nz, 103), lambda i: (0, 0, 0)),
            pl.BlockSpec((E, nz), lambda i: (0, 0)),
            pl.BlockSpec((E, nz, 103), lambda i: (0, 0, 0)),
            pl.BlockSpec((E, nz), lambda i: (0, 0)),
        ],
        out_specs=pl.BlockSpec((2, nz, hb, w), lambda i: (0, 0, i, 0)),
        out_shape=jax.ShapeDtypeStruct((2, nz, h, w), jnp.float32),
        scratch_shapes=[pltpu.VMEM((E * EP, FX), jnp.float32)],
    )(QT, SLI, SST, SOLIN, eta,
      W_base, b_base.reshape(1, 2 * nz),
      coef_qt, int_qt, coef_sli, int_sli)
    return out


# speedup vs baseline: 1.0112x; 1.0112x over previous
import jax
import jax.numpy as jnp
from jax.experimental import pallas as pl
from jax.experimental.pallas import tpu as pltpu

NZ = 34
E = 8
MAX_QT = 15
MAX_SLI = 18
SCALE = 1.0
EP = 72
FX = 80
BT = 8192


def _fold_weights(wb, bb, cq, cs, iq, isl):
    npred = MAX_QT + MAX_SLI
    coef_cat = jnp.concatenate([cq, cs], axis=1)
    int_cat = jnp.concatenate([iq, isl], axis=1)
    A = coef_cat[:, :, :npred]
    W_sel = jnp.concatenate([wb[:MAX_QT], wb[NZ:NZ + MAX_SLI]], axis=0)
    b_sel = jnp.concatenate([bb[:, :MAX_QT], bb[:, NZ:NZ + MAX_SLI]], axis=1)
    D = jnp.concatenate([
        coef_cat[:, :, npred + 1:npred + 1 + NZ],
        coef_cat[:, :, npred + 1 + NZ:npred + 1 + 2 * NZ],
        coef_cat[:, :, npred:npred + 1],
        coef_cat[:, :, npred + 1 + 2 * NZ:],
    ], axis=2)
    M = jax.lax.dot_general(
        A.reshape(E * 2 * NZ, npred), W_sel,
        (((1,), (0,)), ((), ())),
        preferred_element_type=jnp.float32).reshape(E, 2 * NZ, 70)
    R = wb[None] + SCALE * (M + D)
    g = bb + SCALE * (jnp.sum(A * b_sel[:, None, :], axis=-1) + int_cat)
    return R, g


def _routed_kernel(qt_ref, sli_ref, sst_ref, sol_ref, eta_ref,
                   wb_ref, bb_ref, cq_ref, iq_ref, cs_ref, is_ref,
                   out_ref, gs_ref):
    @pl.when(pl.program_id(0) == 0)
    def _fold():
        R, g = _fold_weights(wb_ref[...], bb_ref[...], cq_ref[...],
                             cs_ref[...], iq_ref[...], is_ref[...])
        gs_ref[...] = jnp.zeros((E * EP, FX), jnp.float32)
        for e in range(E):
            gs_ref[e * EP:e * EP + 2 * NZ, 0:NZ] = R[e, :, 0:NZ]
            gs_ref[e * EP:e * EP + 2 * NZ, 40:40 + NZ] = R[e, :, NZ:2 * NZ]
            gs_ref[e * EP:e * EP + 2 * NZ, 74:76] = R[e, :, 2 * NZ:]
            gs_ref[e * EP:e * EP + 2 * NZ, 76:77] = g[e, :, None]

    bt = qt_ref.shape[1] * qt_ref.shape[2]
    x = jnp.concatenate([
        qt_ref[...].reshape(NZ, bt),
        jnp.zeros((40 - NZ, bt), jnp.float32),
        sli_ref[...].reshape(NZ, bt),
        sst_ref[...].reshape(1, bt),
        sol_ref[...].reshape(1, bt),
        jnp.ones((1, bt), jnp.float32),
        jnp.zeros((FX - 77, bt), jnp.float32),
    ], axis=0)
    y = jax.lax.dot_general(
        gs_ref[...], x, (((1,), (0,)), ((), ())),
        preferred_element_type=jnp.float32)
    eta = eta_ref[...].reshape(1, bt)
    b0 = (eta & 1) == 1
    b1 = (eta & 2) == 2
    b2 = (eta & 4) == 4
    z = [jnp.where(b0, y[(2 * k + 1) * EP:(2 * k + 2) * EP, :],
                   y[2 * k * EP:(2 * k + 1) * EP, :]) for k in range(4)]
    q0 = jnp.where(b1, z[1], z[0])
    q1 = jnp.where(b1, z[3], z[2])
    acc = jnp.where(b2, q1, q0)
    out_ref[...] = acc[:2 * NZ, :].reshape(out_ref.shape)


def kernel(QT, SLI, SST, SOLIN, layer_mass, eta, W_base, b_base,
           coef_qt, int_qt, coef_sli, int_sli):
    nz, h, w = QT.shape
    n = h * w
    hb = BT // w
    out = pl.pallas_call(
        _routed_kernel,
        grid=(n // BT,),
        in_specs=[
            pl.BlockSpec((nz, hb, w), lambda i: (0, i, 0)),
            pl.BlockSpec((nz, hb, w), lambda i: (0, i, 0)),
            pl.BlockSpec((hb, w), lambda i: (i, 0)),
            pl.BlockSpec((hb, w), lambda i: (i, 0)),
            pl.BlockSpec((hb, w), lambda i: (i, 0)),
            pl.BlockSpec((2 * nz, 2 * nz + 2), lambda i: (0, 0)),
            pl.BlockSpec((1, 2 * nz), lambda i: (0, 0)),
            pl.BlockSpec((E, nz, 103), lambda i: (0, 0, 0)),
            pl.BlockSpec((E, nz), lambda i: (0, 0)),
            pl.BlockSpec((E, nz, 103), lambda i: (0, 0, 0)),
            pl.BlockSpec((E, nz), lambda i: (0, 0)),
        ],
        out_specs=pl.BlockSpec((2, nz, hb, w), lambda i: (0, 0, i, 0)),
        out_shape=jax.ShapeDtypeStruct((2, nz, h, w), jnp.float32),
        scratch_shapes=[pltpu.VMEM((E * EP, FX), jnp.float32)],
    )(QT, SLI, SST, SOLIN, eta,
      W_base, b_base.reshape(1, 2 * nz),
      coef_qt, int_qt, coef_sli, int_sli)
    return out
